# baseline (speedup 1.0000x reference)
import functools

import jax
import jax.numpy as jnp
from jax.experimental import pallas as pl
from jax.experimental.pallas import tpu as pltpu

_PATCH = 16
_HID = 16
_IMGS = 4


def _fused_body(p_ref, wp_ref, bp_ref,
                w1a_ref, b1a_ref, w1b_ref, b1b_ref,
                w2a_ref, b2a_ref, w2b_ref, b2b_ref,
                out_ref, *, patch_scale, pixel_scale):
    _, n, d = out_ref.shape
    ones_rm = jnp.full((d, 1), 1.0 / d, jnp.float32)
    ones_cm = jnp.full((1, n), 1.0 / n, jnp.float32)

    tok_all = jnp.dot(p_ref[...], wp_ref[...],
                      preferred_element_type=jnp.float32) + bp_ref[...]

    for u in range(_IMGS):
        tok = tok_all[u * n:(u + 1) * n, :]

        row_mean = jnp.dot(tok, ones_rm,
                           preferred_element_type=jnp.float32)
        col_mean = jnp.dot(ones_cm, tok,
                           preferred_element_type=jnp.float32)

        h1 = jnp.maximum(jnp.dot(w1a_ref[...], row_mean,
                                 preferred_element_type=jnp.float32)
                         + b1a_ref[...], 0.0)
        se1 = jax.nn.sigmoid(jnp.dot(w1b_ref[...], h1,
                                     preferred_element_type=jnp.float32)
                             + b1b_ref[...])

        h2 = jnp.maximum(jnp.dot(col_mean, w2a_ref[...],
                                 preferred_element_type=jnp.float32)
                         + b2a_ref[...], 0.0)
        se2 = jax.nn.sigmoid(jnp.dot(h2, w2b_ref[...],
                                     preferred_element_type=jnp.float32)
                             + b2b_ref[...])

        out_ref[u] = tok * (1.0 + patch_scale * se1 + pixel_scale * se2)


def kernel(x, wp, bp, w1a, b1a, w1b, b1b, w2a, b2a, w2b, b2b):
    B, C, H, W = x.shape
    nh, nw = H // _PATCH, W // _PATCH
    n = nh * nw
    pdim = C * _PATCH * _PATCH
    D = wp.shape[1]

    p = x.reshape(B, C, nh, _PATCH, nw, _PATCH)
    p = jnp.transpose(p, (0, 2, 4, 1, 3, 5)).reshape(B * n, pdim)
    p = p.astype(jnp.bfloat16)
    wp_b = wp.astype(jnp.bfloat16)

    body = functools.partial(_fused_body, patch_scale=1.0, pixel_scale=1.0)

    flops_per_img = 2 * n * pdim * D + 4 * n * D + 4 * n * _HID + 4 * D * _HID
    cost = pl.CostEstimate(
        flops=B * flops_per_img,
        transcendentals=B * (n + D),
        bytes_accessed=2 * (B * n * pdim + pdim * D) + 4 * B * n * D,
    )

    steps = B // _IMGS
    return pl.pallas_call(
        body,
        out_shape=jax.ShapeDtypeStruct((B, n, D), jnp.float32),
        grid=(steps,),
        in_specs=[
            pl.BlockSpec((_IMGS * n, pdim), lambda b: (b, 0)),
            pl.BlockSpec((pdim, D), lambda b: (0, 0)),
            pl.BlockSpec((1, D), lambda b: (0, 0)),
            pl.BlockSpec((_HID, n), lambda b: (0, 0)),
            pl.BlockSpec((_HID, 1), lambda b: (0, 0)),
            pl.BlockSpec((n, _HID), lambda b: (0, 0)),
            pl.BlockSpec((n, 1), lambda b: (0, 0)),
            pl.BlockSpec((D, _HID), lambda b: (0, 0)),
            pl.BlockSpec((1, _HID), lambda b: (0, 0)),
            pl.BlockSpec((_HID, D), lambda b: (0, 0)),
            pl.BlockSpec((1, D), lambda b: (0, 0)),
        ],
        out_specs=pl.BlockSpec((_IMGS, n, D), lambda b: (b, 0, 0)),
        compiler_params=pltpu.CompilerParams(
            dimension_semantics=("arbitrary",)),
        cost_estimate=cost,
    )(p, wp_b, bp,
      w1a, b1a, w1b, b1b,
      w2a, b2a, w2b, b2b)


# speedup vs baseline: 2.3185x; 2.3185x over previous
import functools

import jax
import jax.numpy as jnp
from jax.experimental import pallas as pl
from jax.experimental.pallas import tpu as pltpu

_PATCH = 16
_HID = 16
_IMGS = 4


def _fused_body(p_ref, wp_ref, bp_ref,
                w1a_ref, b1a_ref, w1b_ref, b1b_ref,
                w2a_ref, b2a_ref, w2b_ref, b2b_ref,
                out_ref, *, patch_scale, pixel_scale):
    _, n, d = out_ref.shape
    ones_rm = jnp.full((d, 1), 1.0 / d, jnp.float32)
    ones_cm = jnp.full((1, n), 1.0 / n, jnp.float32)

    toks = []
    rms = []
    cms = []
    for u in range(_IMGS):
        tok = jnp.dot(p_ref[u], wp_ref[...],
                      preferred_element_type=jnp.float32) + bp_ref[...]
        toks.append(tok)
        rms.append(jnp.dot(tok, ones_rm,
                           preferred_element_type=jnp.float32))
        cms.append(jnp.dot(ones_cm, tok,
                           preferred_element_type=jnp.float32))

    rm_all = jnp.concatenate(rms, axis=0)
    h1 = jnp.maximum(jnp.dot(w1a_ref[...], rm_all,
                             preferred_element_type=jnp.float32)
                     + b1a_ref[...], 0.0)
    se1_all = jax.nn.sigmoid(jnp.dot(w1b_ref[...], h1,
                                     preferred_element_type=jnp.float32)
                             + b1b_ref[...])

    cm_all = jnp.concatenate(cms, axis=0)
    h2 = jnp.maximum(jnp.dot(cm_all, w2a_ref[...],
                             preferred_element_type=jnp.float32)
                     + b2a_ref[...], 0.0)
    se2_all = jax.nn.sigmoid(jnp.dot(h2, w2b_ref[...],
                                     preferred_element_type=jnp.float32)
                             + b2b_ref[...])

    for u in range(_IMGS):
        se1 = se1_all[u * n:(u + 1) * n, :]
        se2 = se2_all[u:u + 1, :]
        out_ref[u] = toks[u] * (1.0 + patch_scale * se1 + pixel_scale * se2)


def kernel(x, wp, bp, w1a, b1a, w1b, b1b, w2a, b2a, w2b, b2b):
    B, C, H, W = x.shape
    nh, nw = H // _PATCH, W // _PATCH
    n = nh * nw
    pdim = C * _PATCH * _PATCH
    D = wp.shape[1]

    p = x.reshape(B, C, nh, _PATCH, nw, _PATCH)
    p = jnp.transpose(p, (0, 2, 4, 1, 3, 5)).reshape(B, n, pdim)
    p = p.astype(jnp.bfloat16)
    wp_b = wp.astype(jnp.bfloat16)

    eye = jnp.eye(_IMGS, dtype=jnp.float32)
    w1a_bd = jnp.kron(eye, w1a)
    b1a_t = jnp.tile(b1a, (_IMGS, 1))
    w1b_bd = jnp.kron(eye, w1b)
    b1b_t = jnp.tile(b1b, (_IMGS, 1))
    b2a_t = jnp.tile(b2a, (_IMGS, 1))
    b2b_t = jnp.tile(b2b, (_IMGS, 1))

    body = functools.partial(_fused_body, patch_scale=1.0, pixel_scale=1.0)

    flops_per_img = 2 * n * pdim * D + 4 * n * D + 4 * n * _HID + 4 * D * _HID
    cost = pl.CostEstimate(
        flops=B * flops_per_img,
        transcendentals=B * (n + D),
        bytes_accessed=2 * (B * n * pdim + pdim * D) + 4 * B * n * D,
    )

    steps = B // _IMGS
    return pl.pallas_call(
        body,
        out_shape=jax.ShapeDtypeStruct((B, n, D), jnp.float32),
        grid=(steps,),
        in_specs=[
            pl.BlockSpec((_IMGS, n, pdim), lambda b: (b, 0, 0)),
            pl.BlockSpec((pdim, D), lambda b: (0, 0)),
            pl.BlockSpec((1, D), lambda b: (0, 0)),
            pl.BlockSpec((_IMGS * _HID, _IMGS * n), lambda b: (0, 0)),
            pl.BlockSpec((_IMGS * _HID, 1), lambda b: (0, 0)),
            pl.BlockSpec((_IMGS * n, _IMGS * _HID), lambda b: (0, 0)),
            pl.BlockSpec((_IMGS * n, 1), lambda b: (0, 0)),
            pl.BlockSpec((D, _HID), lambda b: (0, 0)),
            pl.BlockSpec((_IMGS, _HID), lambda b: (0, 0)),
            pl.BlockSpec((_HID, D), lambda b: (0, 0)),
            pl.BlockSpec((_IMGS, D), lambda b: (0, 0)),
        ],
        out_specs=pl.BlockSpec((_IMGS, n, D), lambda b: (b, 0, 0)),
        compiler_params=pltpu.CompilerParams(
            dimension_semantics=("arbitrary",)),
        cost_estimate=cost,
    )(p, wp_b, bp,
      w1a_bd, b1a_t, w1b_bd, b1b_t,
      w2a, b2a_t, w2b, b2b_t)


# speedup vs baseline: 2.3512x; 1.0141x over previous
import functools

import jax
import jax.numpy as jnp
from jax.experimental import pallas as pl
from jax.experimental.pallas import tpu as pltpu

_PATCH = 16
_HID = 16
_IMGS = 8


def _fused_body(p_ref, wp_ref, bp_ref,
                w1a_ref, b1a_ref, w1b_ref, b1b_ref,
                w2a_ref, b2a_ref, w2b_ref, b2b_ref,
                out_ref, *, patch_scale, pixel_scale):
    _, n, d = out_ref.shape
    ones_rm = jnp.full((d, 1), 1.0 / d, jnp.float32)
    ones_cm = jnp.full((1, n), 1.0 / n, jnp.float32)

    toks = []
    rms = []
    cms = []
    for u in range(_IMGS):
        tok = jnp.dot(p_ref[u], wp_ref[...],
                      preferred_element_type=jnp.float32) + bp_ref[...]
        toks.append(tok)
        rms.append(jnp.dot(tok, ones_rm,
                           preferred_element_type=jnp.float32))
        cms.append(jnp.dot(ones_cm, tok,
                           preferred_element_type=jnp.float32))

    rm_all = jnp.concatenate(rms, axis=0)
    h1 = jnp.maximum(jnp.dot(w1a_ref[...], rm_all,
                             preferred_element_type=jnp.float32)
                     + b1a_ref[...], 0.0)
    se1_all = jax.nn.sigmoid(jnp.dot(w1b_ref[...], h1,
                                     preferred_element_type=jnp.float32)
                             + b1b_ref[...])

    cm_all = jnp.concatenate(cms, axis=0)
    h2 = jnp.maximum(jnp.dot(cm_all, w2a_ref[...],
                             preferred_element_type=jnp.float32)
                     + b2a_ref[...], 0.0)
    se2_all = jax.nn.sigmoid(jnp.dot(h2, w2b_ref[...],
                                     preferred_element_type=jnp.float32)
                             + b2b_ref[...])

    for u in range(_IMGS):
        se1 = se1_all[u * n:(u + 1) * n, :]
        se2 = se2_all[u:u + 1, :]
        out_ref[u] = toks[u] * (1.0 + patch_scale * se1 + pixel_scale * se2)


def kernel(x, wp, bp, w1a, b1a, w1b, b1b, w2a, b2a, w2b, b2b):
    B, C, H, W = x.shape
    nh, nw = H // _PATCH, W // _PATCH
    n = nh * nw
    pdim = C * _PATCH * _PATCH
    D = wp.shape[1]

    p = x.reshape(B, C, nh, _PATCH, nw, _PATCH)
    p = jnp.transpose(p, (0, 2, 4, 1, 3, 5)).reshape(B, n, pdim)
    p = p.astype(jnp.bfloat16)
    wp_b = wp.astype(jnp.bfloat16)

    eye = jnp.eye(_IMGS, dtype=jnp.float32)
    w1a_bd = jnp.kron(eye, w1a)
    b1a_t = jnp.tile(b1a, (_IMGS, 1))
    w1b_bd = jnp.kron(eye, w1b)
    b1b_t = jnp.tile(b1b, (_IMGS, 1))
    b2a_t = jnp.tile(b2a, (_IMGS, 1))
    b2b_t = jnp.tile(b2b, (_IMGS, 1))

    body = functools.partial(_fused_body, patch_scale=1.0, pixel_scale=1.0)

    flops_per_img = 2 * n * pdim * D + 4 * n * D + 4 * n * _HID + 4 * D * _HID
    cost = pl.CostEstimate(
        flops=B * flops_per_img,
        transcendentals=B * (n + D),
        bytes_accessed=2 * (B * n * pdim + pdim * D) + 4 * B * n * D,
    )

    steps = B // _IMGS
    return pl.pallas_call(
        body,
        out_shape=jax.ShapeDtypeStruct((B, n, D), jnp.float32),
        grid=(steps,),
        in_specs=[
            pl.BlockSpec((_IMGS, n, pdim), lambda b: (b, 0, 0)),
            pl.BlockSpec((pdim, D), lambda b: (0, 0)),
            pl.BlockSpec((1, D), lambda b: (0, 0)),
            pl.BlockSpec((_IMGS * _HID, _IMGS * n), lambda b: (0, 0)),
            pl.BlockSpec((_IMGS * _HID, 1), lambda b: (0, 0)),
            pl.BlockSpec((_IMGS * n, _IMGS * _HID), lambda b: (0, 0)),
            pl.BlockSpec((_IMGS * n, 1), lambda b: (0, 0)),
            pl.BlockSpec((D, _HID), lambda b: (0, 0)),
            pl.BlockSpec((_IMGS, _HID), lambda b: (0, 0)),
            pl.BlockSpec((_HID, D), lambda b: (0, 0)),
            pl.BlockSpec((_IMGS, D), lambda b: (0, 0)),
        ],
        out_specs=pl.BlockSpec((_IMGS, n, D), lambda b: (b, 0, 0)),
        compiler_params=pltpu.CompilerParams(
            dimension_semantics=("arbitrary",)),
        cost_estimate=cost,
    )(p, wp_b, bp,
      w1a_bd, b1a_t, w1b_bd, b1b_t,
      w2a, b2a_t, w2b, b2b_t)
